# baseline (device time: 159081 ns/iter reference)
import jax
import jax.numpy as jnp
from jax import lax
from jax.experimental import pallas as pl
from jax.experimental.pallas import tpu as pltpu

N_DEV = 4
SQ_PER = 256
SQ = N_DEV * SQ_PER
SKV = 4096
H_PER = 8
HQ = 32
DH = 128
D_MODEL = 1024
SCALE = 0.08838834764831843
GW = 128
WW = 512
NGLOB = 32

_MESH = pl.DeviceIdType.MESH


def _mm(a, b, dims):
    return lax.dot_general(a, b, (dims, ((), ())),
                           preferred_element_type=jnp.float32)


def kernel(x, Wq, K_ext, V_ext, Wo):
    x2 = x[0]
    k2 = K_ext[0].reshape(SKV, HQ * DH)
    v2 = V_ext[0].reshape(SKV, HQ * DH)

    def body(x_ref, wq_ref, k2_ref, v2_ref, wo_ref, out_ref,
             xg_ref, ps_ref, pr_ref, wqb_ref, wob_ref,
             kb_ref, vb_ref, kvf_ref,
             kv_sems, xs_sems, xr_sems, ps_sems, pr_sems):
        my_pos = lax.axis_index("i")

        col0 = my_pos * (H_PER * DH)
        strips = ([(k2_ref, kb_ref, h) for h in range(H_PER)]
                  + [(v2_ref, vb_ref, h) for h in range(H_PER)])
        kv_copies = [None] * len(strips)

        def start_strip(s):
            src, _, h = strips[s]
            off = pl.multiple_of(col0 + h * DH, 128)
            cp = pltpu.make_async_copy(
                src.at[:, pl.ds(off, DH)], kvf_ref.at[s % 4],
                kv_sems.at[s % 4])
            cp.start()
            kv_copies[s] = cp

        for s in range(4):
            start_strip(s)

        bar = pltpu.get_barrier_semaphore()
        for d in range(1, N_DEV):
            peer = lax.rem(my_pos + d, N_DEV)
            pl.semaphore_signal(bar, inc=1, device_id=(peer,),
                                device_id_type=_MESH)
        pl.semaphore_wait(bar, N_DEV - 1)

        x16 = x_ref[...].astype(jnp.bfloat16)
        xg_ref[pl.ds(my_pos, 1)] = x16.reshape(1, SQ_PER, D_MODEL)
        x_sends = []
        for d in range(1, N_DEV):
            peer = lax.rem(my_pos + d, N_DEV)
            rdma = pltpu.make_async_remote_copy(
                src_ref=xg_ref.at[pl.ds(my_pos, 1)],
                dst_ref=xg_ref.at[pl.ds(my_pos, 1)],
                send_sem=xs_sems.at[d - 1],
                recv_sem=xr_sems.at[my_pos],
                device_id=(peer,),
                device_id_type=_MESH,
            )
            rdma.start()
            x_sends.append(rdma)

        wqb_ref[...] = wq_ref[...].astype(jnp.bfloat16)
        wob_ref[...] = wo_ref[...].astype(jnp.bfloat16)
        wq_v = wqb_ref[...]

        def qproj(xblk, rows):
            qf = _mm(xblk, wq_v, (((1,), (0,))))
            return qf.astype(jnp.bfloat16).reshape(rows, H_PER, DH)

        q_my = qproj(x16, SQ_PER)

        for s in range(len(strips)):
            kv_copies[s].wait()
            _, dstb, h = strips[s]
            dstb[h] = kvf_ref[s % 4].astype(jnp.bfloat16)
            if s + 4 < len(strips):
                start_strip(s + 4)

        def attend_block(qb3, q0):
            wstart = pl.multiple_of(jnp.maximum(GW, q0 - 128), 128)
            qi_l = lax.broadcasted_iota(jnp.int32, (SQ_PER, GW), 0) + q0
            ki_l = lax.broadcasted_iota(jnp.int32, (SQ_PER, GW), 1)
            mask_low = ((jnp.abs(qi_l - ki_l) <= 128) | (ki_l < NGLOB)
                        | (qi_l < NGLOB))
            qi_w = lax.broadcasted_iota(jnp.int32, (SQ_PER, WW), 0) + q0
            ki_w = lax.broadcasted_iota(jnp.int32, (SQ_PER, WW), 1) + wstart
            mask_win = (jnp.abs(qi_w - ki_w) <= 128) | (qi_w < NGLOB)

            acc = jnp.zeros((SQ_PER, D_MODEL), jnp.float32)
            for h in range(H_PER):
                qh = qb3[:, h, :]
                k_low = kb_ref[h, 0:GW, :]
                k_win = kb_ref[h, pl.ds(wstart, WW), :]
                s_low = _mm(qh, k_low, (((1,), (1,)))) * SCALE
                s_win = _mm(qh, k_win, (((1,), (1,)))) * SCALE
                s_low = jnp.where(mask_low, s_low, -1e9)
                s_win = jnp.where(mask_win, s_win, -1e9)
                m = jnp.maximum(jnp.max(s_low, axis=1, keepdims=True),
                                jnp.max(s_win, axis=1, keepdims=True))
                w_low = jnp.exp(s_low - m)
                w_win = jnp.exp(s_win - m)
                den = (jnp.sum(w_low, axis=1, keepdims=True)
                       + jnp.sum(w_win, axis=1, keepdims=True))
                wl = (w_low / den).astype(jnp.bfloat16)
                ww = (w_win / den).astype(jnp.bfloat16)
                v_low = vb_ref[h, 0:GW, :]
                v_win = vb_ref[h, pl.ds(wstart, WW), :]
                ctx = (_mm(wl, v_low, (((1,), (0,))))
                       + _mm(ww, v_win, (((1,), (0,))))).astype(jnp.bfloat16)
                woh = wob_ref[h * DH:(h + 1) * DH, :]
                acc = acc + _mm(ctx, woh, (((1,), (0,))))
            return acc

        acc_my = attend_block(q_my, my_pos * SQ_PER)
        ps_ref[pl.ds(my_pos, 1)] = (
            acc_my.reshape(1, SQ_PER, D_MODEL).astype(jnp.bfloat16))

        for d in range(1, N_DEV):
            j = lax.rem(my_pos + d, N_DEV)
            recv = pltpu.make_async_remote_copy(
                src_ref=xg_ref.at[pl.ds(j, 1)],
                dst_ref=xg_ref.at[pl.ds(j, 1)],
                send_sem=xs_sems.at[d - 1],
                recv_sem=xr_sems.at[j],
                device_id=(j,),
                device_id_type=_MESH,
            )
            recv.wait_recv()

        q32 = qproj(xg_ref[0, 0:NGLOB, :], NGLOB)

        acc_g = jnp.zeros((NGLOB, D_MODEL), jnp.float32)
        for h in range(H_PER):
            qh = q32[:, h, :]
            s = _mm(qh, kb_ref[h], (((1,), (1,)))) * SCALE
            m = jnp.max(s, axis=1, keepdims=True)
            w = jnp.exp(s - m)
            w = (w / jnp.sum(w, axis=1, keepdims=True)).astype(jnp.bfloat16)
            ctx = _mm(w, vb_ref[h], (((1,), (0,)))).astype(jnp.bfloat16)
            acc_g = acc_g + _mm(ctx, wob_ref[h * DH:(h + 1) * DH, :],
                                (((1,), (0,))))
        acc_g16 = acc_g.astype(jnp.bfloat16)

        @pl.when(my_pos == 0)
        def _():
            ps_ref[0, 0:NGLOB, :] = acc_g16

        p_sends = []
        for d in range(1, N_DEV):
            b = lax.rem(my_pos + d, N_DEV)
            xb = xg_ref[pl.ds(b, 1)].reshape(SQ_PER, D_MODEL)
            acc_b = attend_block(qproj(xb, SQ_PER), b * SQ_PER)
            ps_ref[pl.ds(b, 1)] = (
                acc_b.reshape(1, SQ_PER, D_MODEL).astype(jnp.bfloat16))

            @pl.when(b == 0)
            def _():
                ps_ref[0, 0:NGLOB, :] = acc_g16

            rdma = pltpu.make_async_remote_copy(
                src_ref=ps_ref.at[pl.ds(b, 1)],
                dst_ref=pr_ref.at[pl.ds(my_pos, 1)],
                send_sem=ps_sems.at[d - 1],
                recv_sem=pr_sems.at[my_pos],
                device_id=(b,),
                device_id_type=_MESH,
            )
            rdma.start()
            p_sends.append(rdma)

        pr_ref[pl.ds(my_pos, 1)] = ps_ref[pl.ds(my_pos, 1)]
        for d in range(1, N_DEV):
            j = lax.rem(my_pos + d, N_DEV)
            recv = pltpu.make_async_remote_copy(
                src_ref=ps_ref.at[pl.ds(j, 1)],
                dst_ref=pr_ref.at[pl.ds(j, 1)],
                send_sem=ps_sems.at[d - 1],
                recv_sem=pr_sems.at[j],
                device_id=(j,),
                device_id_type=_MESH,
            )
            recv.wait_recv()

        out_ref[...] = jnp.sum(pr_ref[...].astype(jnp.float32), axis=0)

        for rdma in x_sends + p_sends:
            rdma.wait_send()

    out = pl.pallas_call(
        body,
        out_shape=jax.ShapeDtypeStruct((SQ_PER, D_MODEL), jnp.float32),
        in_specs=[
            pl.BlockSpec(memory_space=pltpu.VMEM),
            pl.BlockSpec(memory_space=pltpu.VMEM),
            pl.BlockSpec(memory_space=pltpu.MemorySpace.HBM),
            pl.BlockSpec(memory_space=pltpu.MemorySpace.HBM),
            pl.BlockSpec(memory_space=pltpu.VMEM),
        ],
        out_specs=pl.BlockSpec(memory_space=pltpu.VMEM),
        scratch_shapes=[
            pltpu.VMEM((N_DEV, SQ_PER, D_MODEL), jnp.bfloat16),
            pltpu.VMEM((N_DEV, SQ_PER, D_MODEL), jnp.bfloat16),
            pltpu.VMEM((N_DEV, SQ_PER, D_MODEL), jnp.bfloat16),
            pltpu.VMEM((D_MODEL, H_PER * DH), jnp.bfloat16),
            pltpu.VMEM((H_PER * DH, D_MODEL), jnp.bfloat16),
            pltpu.VMEM((H_PER, SKV, DH), jnp.bfloat16),
            pltpu.VMEM((H_PER, SKV, DH), jnp.bfloat16),
            pltpu.VMEM((4, SKV, DH), jnp.float32),
            pltpu.SemaphoreType.DMA((4,)),
            pltpu.SemaphoreType.DMA((N_DEV - 1,)),
            pltpu.SemaphoreType.DMA((N_DEV,)),
            pltpu.SemaphoreType.DMA((N_DEV - 1,)),
            pltpu.SemaphoreType.DMA((N_DEV,)),
        ],
        compiler_params=pltpu.CompilerParams(
            collective_id=0, vmem_limit_bytes=100 * 1024 * 1024),
    )(x2, Wq, k2, v2, Wo)
    return out.reshape(1, SQ_PER, D_MODEL)


# device time: 64657 ns/iter; 2.4604x vs baseline; 2.4604x over previous
import jax
import jax.numpy as jnp
from jax import lax
from jax.experimental import pallas as pl
from jax.experimental.pallas import tpu as pltpu

N_DEV = 4
SQ_PER = 256
SQ = N_DEV * SQ_PER
SKV = 4096
H_PER = 8
HQ = 32
DH = 128
D_MODEL = 1024
SCALE = 0.08838834764831843
GW = 128
WW = 512
NGLOB = 32

_MESH = pl.DeviceIdType.MESH


def _mm(a, b, dims):
    return lax.dot_general(a, b, (dims, ((), ())),
                           preferred_element_type=jnp.float32)


def kernel(x, Wq, K_ext, V_ext, Wo):
    x2 = x[0]
    k2 = K_ext[0]
    v2 = V_ext[0]

    def body(x_ref, wq_ref, k2_ref, v2_ref, wo_ref, out_ref,
             xg_ref, ps_ref, pr_ref, wqb_ref, wob_ref,
             kb_ref, vb_ref, kvf_ref,
             kv_sems, xs_sems, xr_sems, ps_sems, pr_sems):
        my_pos = lax.axis_index("i")

        h0 = my_pos * H_PER
        strips = ([(k2_ref, kb_ref, h) for h in range(H_PER)]
                  + [(v2_ref, vb_ref, h) for h in range(H_PER)])
        kv_copies = [None] * len(strips)

        def start_strip(s):
            src, _, h = strips[s]
            cp = pltpu.make_async_copy(
                src.at[:, h0 + h, :], kvf_ref.at[s % 4],
                kv_sems.at[s % 4])
            cp.start()
            kv_copies[s] = cp

        for s in range(4):
            start_strip(s)

        bar = pltpu.get_barrier_semaphore()
        for d in range(1, N_DEV):
            peer = lax.rem(my_pos + d, N_DEV)
            pl.semaphore_signal(bar, inc=1, device_id=(peer,),
                                device_id_type=_MESH)
        pl.semaphore_wait(bar, N_DEV - 1)

        x16 = x_ref[...].astype(jnp.bfloat16)
        xg_ref[pl.ds(my_pos, 1)] = x16.reshape(1, SQ_PER, D_MODEL)
        x_sends = []
        for d in range(1, N_DEV):
            peer = lax.rem(my_pos + d, N_DEV)
            rdma = pltpu.make_async_remote_copy(
                src_ref=xg_ref.at[pl.ds(my_pos, 1)],
                dst_ref=xg_ref.at[pl.ds(my_pos, 1)],
                send_sem=xs_sems.at[d - 1],
                recv_sem=xr_sems.at[my_pos],
                device_id=(peer,),
                device_id_type=_MESH,
            )
            rdma.start()
            x_sends.append(rdma)

        wqb_ref[...] = wq_ref[...].astype(jnp.bfloat16)
        wob_ref[...] = wo_ref[...].astype(jnp.bfloat16)
        wq_v = wqb_ref[...]

        def qproj(xblk, rows):
            qf = _mm(xblk, wq_v, (((1,), (0,))))
            return qf.astype(jnp.bfloat16).reshape(rows, H_PER, DH)

        q_my = qproj(x16, SQ_PER)

        for s in range(len(strips)):
            kv_copies[s].wait()
            _, dstb, h = strips[s]
            dstb[h] = kvf_ref[s % 4].astype(jnp.bfloat16)
            if s + 4 < len(strips):
                start_strip(s + 4)

        def attend_block(qb3, q0):
            wstart = pl.multiple_of(jnp.maximum(GW, q0 - 128), 128)
            qi_l = lax.broadcasted_iota(jnp.int32, (SQ_PER, GW), 0) + q0
            ki_l = lax.broadcasted_iota(jnp.int32, (SQ_PER, GW), 1)
            mask_low = ((jnp.abs(qi_l - ki_l) <= 128) | (ki_l < NGLOB)
                        | (qi_l < NGLOB))
            qi_w = lax.broadcasted_iota(jnp.int32, (SQ_PER, WW), 0) + q0
            ki_w = lax.broadcasted_iota(jnp.int32, (SQ_PER, WW), 1) + wstart
            mask_win = (jnp.abs(qi_w - ki_w) <= 128) | (qi_w < NGLOB)

            acc = jnp.zeros((SQ_PER, D_MODEL), jnp.float32)
            for h in range(H_PER):
                qh = qb3[:, h, :]
                k_low = kb_ref[h, 0:GW, :]
                k_win = kb_ref[h, pl.ds(wstart, WW), :]
                s_low = _mm(qh, k_low, (((1,), (1,)))) * SCALE
                s_win = _mm(qh, k_win, (((1,), (1,)))) * SCALE
                s_low = jnp.where(mask_low, s_low, -1e9)
                s_win = jnp.where(mask_win, s_win, -1e9)
                m = jnp.maximum(jnp.max(s_low, axis=1, keepdims=True),
                                jnp.max(s_win, axis=1, keepdims=True))
                w_low = jnp.exp(s_low - m)
                w_win = jnp.exp(s_win - m)
                den = (jnp.sum(w_low, axis=1, keepdims=True)
                       + jnp.sum(w_win, axis=1, keepdims=True))
                wl = (w_low / den).astype(jnp.bfloat16)
                ww = (w_win / den).astype(jnp.bfloat16)
                v_low = vb_ref[h, 0:GW, :]
                v_win = vb_ref[h, pl.ds(wstart, WW), :]
                ctx = (_mm(wl, v_low, (((1,), (0,))))
                       + _mm(ww, v_win, (((1,), (0,))))).astype(jnp.bfloat16)
                woh = wob_ref[h * DH:(h + 1) * DH, :]
                acc = acc + _mm(ctx, woh, (((1,), (0,))))
            return acc

        acc_my = attend_block(q_my, my_pos * SQ_PER)
        ps_ref[pl.ds(my_pos, 1)] = (
            acc_my.reshape(1, SQ_PER, D_MODEL).astype(jnp.bfloat16))

        for d in range(1, N_DEV):
            j = lax.rem(my_pos + d, N_DEV)
            recv = pltpu.make_async_remote_copy(
                src_ref=xg_ref.at[pl.ds(j, 1)],
                dst_ref=xg_ref.at[pl.ds(j, 1)],
                send_sem=xs_sems.at[d - 1],
                recv_sem=xr_sems.at[j],
                device_id=(j,),
                device_id_type=_MESH,
            )
            recv.wait_recv()

        q32 = qproj(xg_ref[0, 0:NGLOB, :], NGLOB)

        acc_g = jnp.zeros((NGLOB, D_MODEL), jnp.float32)
        for h in range(H_PER):
            qh = q32[:, h, :]
            s = _mm(qh, kb_ref[h], (((1,), (1,)))) * SCALE
            m = jnp.max(s, axis=1, keepdims=True)
            w = jnp.exp(s - m)
            w = (w / jnp.sum(w, axis=1, keepdims=True)).astype(jnp.bfloat16)
            ctx = _mm(w, vb_ref[h], (((1,), (0,)))).astype(jnp.bfloat16)
            acc_g = acc_g + _mm(ctx, wob_ref[h * DH:(h + 1) * DH, :],
                                (((1,), (0,))))
        acc_g16 = acc_g.astype(jnp.bfloat16)

        @pl.when(my_pos == 0)
        def _():
            ps_ref[0, 0:NGLOB, :] = acc_g16

        p_sends = []
        for d in range(1, N_DEV):
            b = lax.rem(my_pos + d, N_DEV)
            xb = xg_ref[pl.ds(b, 1)].reshape(SQ_PER, D_MODEL)
            acc_b = attend_block(qproj(xb, SQ_PER), b * SQ_PER)
            ps_ref[pl.ds(b, 1)] = (
                acc_b.reshape(1, SQ_PER, D_MODEL).astype(jnp.bfloat16))

            @pl.when(b == 0)
            def _():
                ps_ref[0, 0:NGLOB, :] = acc_g16

            rdma = pltpu.make_async_remote_copy(
                src_ref=ps_ref.at[pl.ds(b, 1)],
                dst_ref=pr_ref.at[pl.ds(my_pos, 1)],
                send_sem=ps_sems.at[d - 1],
                recv_sem=pr_sems.at[my_pos],
                device_id=(b,),
                device_id_type=_MESH,
            )
            rdma.start()
            p_sends.append(rdma)

        pr_ref[pl.ds(my_pos, 1)] = ps_ref[pl.ds(my_pos, 1)]
        for d in range(1, N_DEV):
            j = lax.rem(my_pos + d, N_DEV)
            recv = pltpu.make_async_remote_copy(
                src_ref=ps_ref.at[pl.ds(j, 1)],
                dst_ref=pr_ref.at[pl.ds(j, 1)],
                send_sem=ps_sems.at[d - 1],
                recv_sem=pr_sems.at[j],
                device_id=(j,),
                device_id_type=_MESH,
            )
            recv.wait_recv()

        out_ref[...] = jnp.sum(pr_ref[...].astype(jnp.float32), axis=0)

        for rdma in x_sends + p_sends:
            rdma.wait_send()

    out = pl.pallas_call(
        body,
        out_shape=jax.ShapeDtypeStruct((SQ_PER, D_MODEL), jnp.float32),
        in_specs=[
            pl.BlockSpec(memory_space=pltpu.VMEM),
            pl.BlockSpec(memory_space=pltpu.VMEM),
            pl.BlockSpec(memory_space=pltpu.MemorySpace.HBM),
            pl.BlockSpec(memory_space=pltpu.MemorySpace.HBM),
            pl.BlockSpec(memory_space=pltpu.VMEM),
        ],
        out_specs=pl.BlockSpec(memory_space=pltpu.VMEM),
        scratch_shapes=[
            pltpu.VMEM((N_DEV, SQ_PER, D_MODEL), jnp.bfloat16),
            pltpu.VMEM((N_DEV, SQ_PER, D_MODEL), jnp.bfloat16),
            pltpu.VMEM((N_DEV, SQ_PER, D_MODEL), jnp.bfloat16),
            pltpu.VMEM((D_MODEL, H_PER * DH), jnp.bfloat16),
            pltpu.VMEM((H_PER * DH, D_MODEL), jnp.bfloat16),
            pltpu.VMEM((H_PER, SKV, DH), jnp.bfloat16),
            pltpu.VMEM((H_PER, SKV, DH), jnp.bfloat16),
            pltpu.VMEM((4, SKV, DH), jnp.float32),
            pltpu.SemaphoreType.DMA((4,)),
            pltpu.SemaphoreType.DMA((N_DEV - 1,)),
            pltpu.SemaphoreType.DMA((N_DEV,)),
            pltpu.SemaphoreType.DMA((N_DEV - 1,)),
            pltpu.SemaphoreType.DMA((N_DEV,)),
        ],
        compiler_params=pltpu.CompilerParams(
            collective_id=0, vmem_limit_bytes=100 * 1024 * 1024),
    )(x2, Wq, k2, v2, Wo)
    return out.reshape(1, SQ_PER, D_MODEL)


# device time: 56182 ns/iter; 2.8315x vs baseline; 1.1508x over previous
import jax
import jax.numpy as jnp
from jax import lax
from jax.experimental import pallas as pl
from jax.experimental.pallas import tpu as pltpu

N_DEV = 4
SQ_PER = 256
SQ = N_DEV * SQ_PER
SKV = 4096
H_PER = 8
HQ = 32
DH = 128
D_MODEL = 1024
SCALE = 0.08838834764831843
GW = 128
WW = 512
NGLOB = 32

_MESH = pl.DeviceIdType.MESH


def _mm(a, b, dims):
    return lax.dot_general(a, b, (dims, ((), ())),
                           preferred_element_type=jnp.float32)


def kernel(x, Wq, K_ext, V_ext, Wo):
    x2 = x[0]
    k2 = K_ext[0]
    v2 = V_ext[0]

    def body(x_ref, wq_ref, k2_ref, v2_ref, wo_ref, out_ref,
             xg_ref, ps_ref, pr_ref, wqb_ref, wob_ref,
             kb_ref, vb_ref, kvf_ref,
             kv_sems, xs_sems, xr_sems, ps_sems, pr_sems):
        my_pos = lax.axis_index("i")

        h0 = my_pos * H_PER
        strips = []
        for h in range(H_PER):
            strips.append((k2_ref, kb_ref, h))
            strips.append((v2_ref, vb_ref, h))
        kv_copies = [None] * len(strips)

        def start_strip(s):
            src, _, h = strips[s]
            cp = pltpu.make_async_copy(
                src.at[:, h0 + h, :], kvf_ref.at[s % 4],
                kv_sems.at[s % 4])
            cp.start()
            kv_copies[s] = cp

        for s in range(4):
            start_strip(s)

        bar = pltpu.get_barrier_semaphore()
        for d in range(1, N_DEV):
            peer = lax.rem(my_pos + d, N_DEV)
            pl.semaphore_signal(bar, inc=1, device_id=(peer,),
                                device_id_type=_MESH)
        pl.semaphore_wait(bar, N_DEV - 1)

        x16 = x_ref[...].astype(jnp.bfloat16)
        xg_ref[pl.ds(my_pos, 1)] = x16.reshape(1, SQ_PER, D_MODEL)
        x_sends = []
        for d in range(1, N_DEV):
            peer = lax.rem(my_pos + d, N_DEV)
            rdma = pltpu.make_async_remote_copy(
                src_ref=xg_ref.at[pl.ds(my_pos, 1)],
                dst_ref=xg_ref.at[pl.ds(my_pos, 1)],
                send_sem=xs_sems.at[d - 1],
                recv_sem=xr_sems.at[my_pos],
                device_id=(peer,),
                device_id_type=_MESH,
            )
            rdma.start()
            x_sends.append(rdma)

        wqb_ref[...] = wq_ref[...].astype(jnp.bfloat16)
        wob_ref[...] = wo_ref[...].astype(jnp.bfloat16)
        wq_v = wqb_ref[...]

        def qproj(xblk):
            qf = _mm(xblk, wq_v, (((1,), (0,))))
            return qf.astype(jnp.bfloat16)

        def land_strip(s):
            kv_copies[s].wait()
            _, dstb, h = strips[s]
            dstb[h] = kvf_ref[s % 4].astype(jnp.bfloat16)
            if s + 4 < len(strips):
                start_strip(s + 4)

        q_my = qproj(x16)

        def attend_block(qb, q0, land=False):
            wstart = pl.multiple_of(jnp.maximum(GW, q0 - 128), 128)
            qi_l = lax.broadcasted_iota(jnp.int32, (SQ_PER, GW), 0) + q0
            ki_l = lax.broadcasted_iota(jnp.int32, (SQ_PER, GW), 1)
            mask_low = ((jnp.abs(qi_l - ki_l) <= 128) | (ki_l < NGLOB)
                        | (qi_l < NGLOB))
            qi_w = lax.broadcasted_iota(jnp.int32, (SQ_PER, WW), 0) + q0
            ki_w = lax.broadcasted_iota(jnp.int32, (SQ_PER, WW), 1) + wstart
            mask_win = (jnp.abs(qi_w - ki_w) <= 128) | (qi_w < NGLOB)

            ctxs = []
            for h in range(H_PER):
                if land:
                    land_strip(2 * h)
                    land_strip(2 * h + 1)
                qh = qb[:, h * DH:(h + 1) * DH]
                k_low = kb_ref[h, 0:GW, :]
                k_win = kb_ref[h, pl.ds(wstart, WW), :]
                s_low = _mm(qh, k_low, (((1,), (1,)))) * SCALE
                s_win = _mm(qh, k_win, (((1,), (1,)))) * SCALE
                s_low = jnp.where(mask_low, s_low, -1e9)
                s_win = jnp.where(mask_win, s_win, -1e9)
                m = jnp.maximum(jnp.max(s_low, axis=1, keepdims=True),
                                jnp.max(s_win, axis=1, keepdims=True))
                w_low = jnp.exp(s_low - m)
                w_win = jnp.exp(s_win - m)
                den = (jnp.sum(w_low, axis=1, keepdims=True)
                       + jnp.sum(w_win, axis=1, keepdims=True))
                wl = (w_low / den).astype(jnp.bfloat16)
                ww = (w_win / den).astype(jnp.bfloat16)
                v_low = vb_ref[h, 0:GW, :]
                v_win = vb_ref[h, pl.ds(wstart, WW), :]
                ctxs.append((_mm(wl, v_low, (((1,), (0,))))
                             + _mm(ww, v_win, (((1,), (0,)))))
                            .astype(jnp.bfloat16))
            ctx_all = jnp.concatenate(ctxs, axis=1)
            return _mm(ctx_all, wob_ref[...], (((1,), (0,))))

        acc_my = attend_block(q_my, my_pos * SQ_PER, land=True)
        ps_ref[pl.ds(my_pos, 1)] = (
            acc_my.reshape(1, SQ_PER, D_MODEL).astype(jnp.bfloat16))

        for d in range(1, N_DEV):
            j = lax.rem(my_pos + d, N_DEV)
            recv = pltpu.make_async_remote_copy(
                src_ref=xg_ref.at[pl.ds(j, 1)],
                dst_ref=xg_ref.at[pl.ds(j, 1)],
                send_sem=xs_sems.at[d - 1],
                recv_sem=xr_sems.at[j],
                device_id=(j,),
                device_id_type=_MESH,
            )
            recv.wait_recv()

        q32 = qproj(xg_ref[0, 0:NGLOB, :])

        gctxs = []
        for h in range(H_PER):
            qh = q32[:, h * DH:(h + 1) * DH]
            s = _mm(qh, kb_ref[h], (((1,), (1,)))) * SCALE
            m = jnp.max(s, axis=1, keepdims=True)
            w = jnp.exp(s - m)
            w = (w / jnp.sum(w, axis=1, keepdims=True)).astype(jnp.bfloat16)
            gctxs.append(_mm(w, vb_ref[h], (((1,), (0,)))).astype(jnp.bfloat16))
        acc_g = _mm(jnp.concatenate(gctxs, axis=1), wob_ref[...], (((1,), (0,))))
        acc_g16 = acc_g.astype(jnp.bfloat16)

        @pl.when(my_pos == 0)
        def _():
            ps_ref[0, 0:NGLOB, :] = acc_g16

        p_sends = []
        for d in range(1, N_DEV):
            b = lax.rem(my_pos + d, N_DEV)
            xb = xg_ref[pl.ds(b, 1)].reshape(SQ_PER, D_MODEL)
            acc_b = attend_block(qproj(xb), b * SQ_PER)
            ps_ref[pl.ds(b, 1)] = (
                acc_b.reshape(1, SQ_PER, D_MODEL).astype(jnp.bfloat16))

            @pl.when(b == 0)
            def _():
                ps_ref[0, 0:NGLOB, :] = acc_g16

            rdma = pltpu.make_async_remote_copy(
                src_ref=ps_ref.at[pl.ds(b, 1)],
                dst_ref=pr_ref.at[pl.ds(my_pos, 1)],
                send_sem=ps_sems.at[d - 1],
                recv_sem=pr_sems.at[my_pos],
                device_id=(b,),
                device_id_type=_MESH,
            )
            rdma.start()
            p_sends.append(rdma)

        pr_ref[pl.ds(my_pos, 1)] = ps_ref[pl.ds(my_pos, 1)]
        for d in range(1, N_DEV):
            j = lax.rem(my_pos + d, N_DEV)
            recv = pltpu.make_async_remote_copy(
                src_ref=ps_ref.at[pl.ds(j, 1)],
                dst_ref=pr_ref.at[pl.ds(j, 1)],
                send_sem=ps_sems.at[d - 1],
                recv_sem=pr_sems.at[j],
                device_id=(j,),
                device_id_type=_MESH,
            )
            recv.wait_recv()

        out_ref[...] = jnp.sum(pr_ref[...].astype(jnp.float32), axis=0)

        for rdma in x_sends + p_sends:
            rdma.wait_send()

    out = pl.pallas_call(
        body,
        out_shape=jax.ShapeDtypeStruct((SQ_PER, D_MODEL), jnp.float32),
        in_specs=[
            pl.BlockSpec(memory_space=pltpu.VMEM),
            pl.BlockSpec(memory_space=pltpu.VMEM),
            pl.BlockSpec(memory_space=pltpu.MemorySpace.HBM),
            pl.BlockSpec(memory_space=pltpu.MemorySpace.HBM),
            pl.BlockSpec(memory_space=pltpu.VMEM),
        ],
        out_specs=pl.BlockSpec(memory_space=pltpu.VMEM),
        scratch_shapes=[
            pltpu.VMEM((N_DEV, SQ_PER, D_MODEL), jnp.bfloat16),
            pltpu.VMEM((N_DEV, SQ_PER, D_MODEL), jnp.bfloat16),
            pltpu.VMEM((N_DEV, SQ_PER, D_MODEL), jnp.bfloat16),
            pltpu.VMEM((D_MODEL, H_PER * DH), jnp.bfloat16),
            pltpu.VMEM((H_PER * DH, D_MODEL), jnp.bfloat16),
            pltpu.VMEM((H_PER, SKV, DH), jnp.bfloat16),
            pltpu.VMEM((H_PER, SKV, DH), jnp.bfloat16),
            pltpu.VMEM((4, SKV, DH), jnp.float32),
            pltpu.SemaphoreType.DMA((4,)),
            pltpu.SemaphoreType.DMA((N_DEV - 1,)),
            pltpu.SemaphoreType.DMA((N_DEV,)),
            pltpu.SemaphoreType.DMA((N_DEV - 1,)),
            pltpu.SemaphoreType.DMA((N_DEV,)),
        ],
        compiler_params=pltpu.CompilerParams(
            collective_id=0, vmem_limit_bytes=100 * 1024 * 1024),
    )(x2, Wq, k2, v2, Wo)
    return out.reshape(1, SQ_PER, D_MODEL)


# device time: 51957 ns/iter; 3.0618x vs baseline; 1.0813x over previous
import jax
import jax.numpy as jnp
from jax import lax
from jax.experimental import pallas as pl
from jax.experimental.pallas import tpu as pltpu

N_DEV = 4
SQ_PER = 256
SQ = N_DEV * SQ_PER
SKV = 4096
H_PER = 8
HQ = 32
DH = 128
D_MODEL = 1024
SCALE = 0.08838834764831843
GW = 128
WW = 512
NGLOB = 32

_MESH = pl.DeviceIdType.MESH


def _mm(a, b, dims):
    return lax.dot_general(a, b, (dims, ((), ())),
                           preferred_element_type=jnp.float32)


def kernel(x, Wq, K_ext, V_ext, Wo):
    x2 = x[0]
    k2 = K_ext[0]
    v2 = V_ext[0]

    def body(x_ref, wq_ref, k2_ref, v2_ref, wo_ref, out_ref,
             xg_ref, ps_ref, pr_ref, wqb_ref, wob_ref,
             kb_ref, vb_ref, kvf_ref,
             kv_sems, xs_sems, xr_sems, ps_sems, pr_sems):
        my_pos = lax.axis_index("i")

        h0 = my_pos * H_PER
        strips = []
        for h in range(H_PER):
            strips.append((k2_ref, kb_ref, h))
            strips.append((v2_ref, vb_ref, h))
        kv_copies = [None] * len(strips)

        def start_strip(s):
            src, _, h = strips[s]
            cp = pltpu.make_async_copy(
                src.at[:, h0 + h, :], kvf_ref.at[s % 4],
                kv_sems.at[s % 4])
            cp.start()
            kv_copies[s] = cp

        for s in range(4):
            start_strip(s)

        bar = pltpu.get_barrier_semaphore()
        for d in range(1, N_DEV):
            peer = lax.rem(my_pos + d, N_DEV)
            pl.semaphore_signal(bar, inc=1, device_id=(peer,),
                                device_id_type=_MESH)
        pl.semaphore_wait(bar, N_DEV - 1)

        x16 = x_ref[...].astype(jnp.bfloat16)
        xg_ref[pl.ds(my_pos, 1)] = x16.reshape(1, SQ_PER, D_MODEL)
        x_sends = []
        for d in range(1, N_DEV):
            peer = lax.rem(my_pos + d, N_DEV)
            rdma = pltpu.make_async_remote_copy(
                src_ref=xg_ref.at[pl.ds(my_pos, 1)],
                dst_ref=xg_ref.at[pl.ds(my_pos, 1)],
                send_sem=xs_sems.at[d - 1],
                recv_sem=xr_sems.at[my_pos],
                device_id=(peer,),
                device_id_type=_MESH,
            )
            rdma.start()
            x_sends.append(rdma)

        wqb_ref[...] = wq_ref[...].astype(jnp.bfloat16)
        wob_ref[...] = wo_ref[...].astype(jnp.bfloat16)
        wq_v = wqb_ref[...]

        def qproj(xblk):
            qf = _mm(xblk, wq_v, (((1,), (0,))))
            return qf.astype(jnp.bfloat16)

        def land_strip(s):
            kv_copies[s].wait()
            _, dstb, h = strips[s]
            dstb[h] = kvf_ref[s % 4].astype(jnp.bfloat16)
            if s + 4 < len(strips):
                start_strip(s + 4)

        q_my = qproj(x16)

        def attend_rows(qb, q0, row0, nrows, land=False):
            r0 = q0 + row0
            wstart = pl.multiple_of(jnp.maximum(GW, q0 - 128), 128)
            qi_l = lax.broadcasted_iota(jnp.int32, (nrows, GW), 0) + r0
            ki_l = lax.broadcasted_iota(jnp.int32, (nrows, GW), 1)
            mask_low = ((jnp.abs(qi_l - ki_l) <= 128) | (ki_l < NGLOB)
                        | (qi_l < NGLOB))
            qi_w = lax.broadcasted_iota(jnp.int32, (nrows, WW), 0) + r0
            ki_w = lax.broadcasted_iota(jnp.int32, (nrows, WW), 1) + wstart
            mask_win = (jnp.abs(qi_w - ki_w) <= 128) | (qi_w < NGLOB)

            ctxs = []
            for h in range(H_PER):
                if land:
                    land_strip(2 * h)
                    land_strip(2 * h + 1)
                qh = qb[row0:row0 + nrows, h * DH:(h + 1) * DH]
                k_low = kb_ref[h, 0:GW, :]
                k_win = kb_ref[h, pl.ds(wstart, WW), :]
                s_low = _mm(qh, k_low, (((1,), (1,)))) * SCALE
                s_win = _mm(qh, k_win, (((1,), (1,)))) * SCALE
                w_low = jnp.exp(jnp.where(mask_low, s_low, -1e9))
                w_win = jnp.exp(jnp.where(mask_win, s_win, -1e9))
                den = (jnp.sum(w_low, axis=1, keepdims=True)
                       + jnp.sum(w_win, axis=1, keepdims=True))
                wl = (w_low / den).astype(jnp.bfloat16)
                ww = (w_win / den).astype(jnp.bfloat16)
                v_low = vb_ref[h, 0:GW, :]
                v_win = vb_ref[h, pl.ds(wstart, WW), :]
                ctxs.append((_mm(wl, v_low, (((1,), (0,))))
                             + _mm(ww, v_win, (((1,), (0,)))))
                            .astype(jnp.bfloat16))
            ctx_all = jnp.concatenate(ctxs, axis=1)
            return _mm(ctx_all, wob_ref[...], (((1,), (0,))))

        HALF = SQ_PER // 2
        my_q0 = my_pos * SQ_PER
        acc_my1 = attend_rows(q_my, my_q0, 0, HALF, land=True)
        ps_ref[pl.ds(my_pos, 1), 0:HALF, :] = (
            acc_my1.reshape(1, HALF, D_MODEL).astype(jnp.bfloat16))

        for d in range(1, N_DEV):
            j = lax.rem(my_pos + d, N_DEV)
            recv = pltpu.make_async_remote_copy(
                src_ref=xg_ref.at[pl.ds(j, 1)],
                dst_ref=xg_ref.at[pl.ds(j, 1)],
                send_sem=xs_sems.at[d - 1],
                recv_sem=xr_sems.at[j],
                device_id=(j,),
                device_id_type=_MESH,
            )
            recv.wait_recv()

        q32 = qproj(xg_ref[0, 0:NGLOB, :])

        gctxs = []
        for h in range(H_PER):
            qh = q32[:, h * DH:(h + 1) * DH]
            s = _mm(qh, kb_ref[h], (((1,), (1,)))) * SCALE
            w = jnp.exp(s)
            w = (w / jnp.sum(w, axis=1, keepdims=True)).astype(jnp.bfloat16)
            gctxs.append(_mm(w, vb_ref[h], (((1,), (0,)))).astype(jnp.bfloat16))
        acc_g = _mm(jnp.concatenate(gctxs, axis=1), wob_ref[...], (((1,), (0,))))
        acc_g16 = acc_g.astype(jnp.bfloat16)

        @pl.when(my_pos == 0)
        def _():
            ps_ref[0, 0:NGLOB, :] = acc_g16

        p_sends = []
        for d in range(1, N_DEV):
            b = lax.rem(my_pos + d, N_DEV)
            xb = xg_ref[pl.ds(b, 1)].reshape(SQ_PER, D_MODEL)
            acc_b = attend_rows(qproj(xb), b * SQ_PER, 0, SQ_PER)
            ps_ref[pl.ds(b, 1)] = (
                acc_b.reshape(1, SQ_PER, D_MODEL).astype(jnp.bfloat16))

            @pl.when(b == 0)
            def _():
                ps_ref[0, 0:NGLOB, :] = acc_g16

            rdma = pltpu.make_async_remote_copy(
                src_ref=ps_ref.at[pl.ds(b, 1)],
                dst_ref=pr_ref.at[pl.ds(my_pos, 1)],
                send_sem=ps_sems.at[d - 1],
                recv_sem=pr_sems.at[my_pos],
                device_id=(b,),
                device_id_type=_MESH,
            )
            rdma.start()
            p_sends.append(rdma)

        acc_my2 = attend_rows(q_my, my_q0, HALF, HALF)
        ps_ref[pl.ds(my_pos, 1), HALF:SQ_PER, :] = (
            acc_my2.reshape(1, HALF, D_MODEL).astype(jnp.bfloat16))

        pr_ref[pl.ds(my_pos, 1)] = ps_ref[pl.ds(my_pos, 1)]
        for d in range(1, N_DEV):
            j = lax.rem(my_pos + d, N_DEV)
            recv = pltpu.make_async_remote_copy(
                src_ref=ps_ref.at[pl.ds(j, 1)],
                dst_ref=pr_ref.at[pl.ds(j, 1)],
                send_sem=ps_sems.at[d - 1],
                recv_sem=pr_sems.at[j],
                device_id=(j,),
                device_id_type=_MESH,
            )
            recv.wait_recv()

        out_ref[...] = jnp.sum(pr_ref[...].astype(jnp.float32), axis=0)

        for rdma in x_sends + p_sends:
            rdma.wait_send()

    out = pl.pallas_call(
        body,
        out_shape=jax.ShapeDtypeStruct((SQ_PER, D_MODEL), jnp.float32),
        in_specs=[
            pl.BlockSpec(memory_space=pltpu.VMEM),
            pl.BlockSpec(memory_space=pltpu.VMEM),
            pl.BlockSpec(memory_space=pltpu.MemorySpace.HBM),
            pl.BlockSpec(memory_space=pltpu.MemorySpace.HBM),
            pl.BlockSpec(memory_space=pltpu.VMEM),
        ],
        out_specs=pl.BlockSpec(memory_space=pltpu.VMEM),
        scratch_shapes=[
            pltpu.VMEM((N_DEV, SQ_PER, D_MODEL), jnp.bfloat16),
            pltpu.VMEM((N_DEV, SQ_PER, D_MODEL), jnp.bfloat16),
            pltpu.VMEM((N_DEV, SQ_PER, D_MODEL), jnp.bfloat16),
            pltpu.VMEM((D_MODEL, H_PER * DH), jnp.bfloat16),
            pltpu.VMEM((H_PER * DH, D_MODEL), jnp.bfloat16),
            pltpu.VMEM((H_PER, SKV, DH), jnp.bfloat16),
            pltpu.VMEM((H_PER, SKV, DH), jnp.bfloat16),
            pltpu.VMEM((4, SKV, DH), jnp.float32),
            pltpu.SemaphoreType.DMA((4,)),
            pltpu.SemaphoreType.DMA((N_DEV - 1,)),
            pltpu.SemaphoreType.DMA((N_DEV,)),
            pltpu.SemaphoreType.DMA((N_DEV - 1,)),
            pltpu.SemaphoreType.DMA((N_DEV,)),
        ],
        compiler_params=pltpu.CompilerParams(
            collective_id=0, vmem_limit_bytes=100 * 1024 * 1024),
    )(x2, Wq, k2, v2, Wo)
    return out.reshape(1, SQ_PER, D_MODEL)
